# baseline (speedup 1.0000x reference)
import functools

import jax
import jax.numpy as jnp
from jax import lax
from jax.experimental import pallas as pl
from jax.experimental.pallas import tpu as pltpu
from jax.experimental.pallas import tpu_sc as plsc

_N = 100000
_E = 1600000
_F = 128
_R = _E // 128
_NW = 32
_ROWS = 391
_NPAD = 100096
_CHUNK = _NPAD // 16


_LROWS = 392
_HROWS = _LROWS // 2


def _sc_segment_body(ei_hbm, he_hbm, out_hbm, ei_v, val_v, zbuf, acc):
    c = lax.axis_index("c")
    s = lax.axis_index("s")
    w = s * 2 + c

    base = (w * _R) // _NW
    cnt = ((w + 1) * _R) // _NW - base
    load_base = jnp.minimum(base, _R - _LROWS)
    off = base - load_base

    pltpu.sync_copy(he_hbm.at[pl.ds(load_base * 128, _LROWS * 128)], val_v)

    z16f = jnp.zeros((16,), jnp.float32)

    def zero_body(i, carry):
        zbuf[pl.ds(i * 16, 16)] = z16f
        return carry

    lax.fori_loop(0, _CHUNK // 16, zero_body, 0)
    pltpu.sync_copy(zbuf, acc.at[pl.ds(s * _CHUNK, _CHUNK)])
    plsc.subcore_barrier()

    for h in range(2):
        pltpu.sync_copy(ei_hbm.at[pl.ds(load_base + h * _HROWS, _HROWS)],
                        ei_v)
        j_lo = jnp.maximum(off, h * _HROWS) - h * _HROWS
        j_hi = jnp.minimum(off + cnt, (h + 1) * _HROWS) - h * _HROWS

        def scat_body(j, carry, _h=h):
            pltpu.sync_copy(val_v.at[pl.ds((_h * _HROWS + j) * 128, 128)],
                            acc.at[ei_v.at[j, 1]], add=True)
            return carry

        lax.fori_loop(j_lo, j_hi, scat_body, 0)
    plsc.subcore_barrier()

    pltpu.sync_copy(acc.at[pl.ds(s * _CHUNK, _CHUNK)], zbuf)
    pltpu.sync_copy(zbuf, out_hbm.at[c].at[pl.ds(s * _CHUNK, _CHUNK)])


@functools.cache
def _make_sc_segment():
    return pl.kernel(
        _sc_segment_body,
        out_type=jax.ShapeDtypeStruct((2, _NPAD), jnp.float32),
        mesh=plsc.VectorSubcoreMesh(core_axis_name="c", subcore_axis_name="s"),
        compiler_params=pltpu.CompilerParams(use_tc_tiling_on_sc=False),
        scratch_types=[
            pltpu.VMEM((_HROWS, 2, 128), jnp.int32),
            pltpu.VMEM((_LROWS * 128,), jnp.float32),
            pltpu.VMEM((_CHUNK,), jnp.float32),
            pltpu.VMEM_SHARED((_NPAD,), jnp.float32),
        ],
    )


_BN = 2048


def _mlp_body(pt_ref, hn_ref, w1a_ref, w1b_ref, b1_ref, w2_ref, b2_ref,
              w3t_ref, b3t_ref, out_ref):
    x = hn_ref[...]
    h = jnp.dot(x, w1a_ref[...], preferred_element_type=jnp.float32)
    agg = pt_ref[...].reshape(_BN, 1)
    h = h + agg * w1b_ref[...]
    h = jnp.maximum(h + b1_ref[...], 0.0)
    h = jnp.dot(h, w2_ref[...], preferred_element_type=jnp.float32)
    h = jnp.maximum(h + b2_ref[...], 0.0)
    out_ref[...] = lax.dot_general(
        w3t_ref[...], h, (((1,), (1,)), ((), ())),
        preferred_element_type=jnp.float32,
    ) + b3t_ref[...]


def _mlp_tc(pt, hn, w1a, w1b, b1, w2, b2, w3t, b3t):
    n_out = w3t.shape[0]
    grid = (pl.cdiv(_N, _BN),)
    return pl.pallas_call(
        _mlp_body,
        grid=grid,
        in_specs=[
            pl.BlockSpec((_BN,), lambda i: (i,)),
            pl.BlockSpec((_BN, _F), lambda i: (i, 0)),
            pl.BlockSpec((_F, _F), lambda i: (0, 0)),
            pl.BlockSpec((1, _F), lambda i: (0, 0)),
            pl.BlockSpec((1, _F), lambda i: (0, 0)),
            pl.BlockSpec((_F, _F), lambda i: (0, 0)),
            pl.BlockSpec((1, _F), lambda i: (0, 0)),
            pl.BlockSpec((n_out, _F), lambda i: (0, 0)),
            pl.BlockSpec((n_out, 1), lambda i: (0, 0)),
        ],
        out_specs=pl.BlockSpec((n_out, _BN), lambda i: (0, i)),
        out_shape=jax.ShapeDtypeStruct((n_out, _N), jnp.float32),
    )(pt, hn, w1a, w1b, b1, w2, b2, w3t, b3t)


def kernel(hn, he, edge_index, W1, b1, W2, b2, W3, b3):
    ei3 = edge_index.astype(jnp.int32).reshape(2, _R, 128).transpose(1, 0, 2)
    he1 = he.reshape(_E)
    partials = _make_sc_segment()(ei3, he1)
    pt = partials[0, :_N] + partials[1, :_N]

    w1a = W1[:_F]
    w1b = W1[_F:_F + 1]
    out3 = _mlp_tc(
        pt, hn, w1a, w1b, b1.reshape(1, _F), W2, b2.reshape(1, _F),
        W3.T, b3.reshape(-1, 1),
    )
    return out3.T


# speedup vs baseline: 1.1569x; 1.1569x over previous
import functools

import jax
import jax.numpy as jnp
from jax import lax
from jax.experimental import pallas as pl
from jax.experimental.pallas import tpu as pltpu
from jax.experimental.pallas import tpu_sc as plsc

_N = 100000
_E = 1600000
_F = 128
_R = _E // 128
_NW = 32
_NPAD = 100096
_CHUNK = _NPAD // 16

_LROWS = 392
_HROWS = _LROWS // 2


def _zero_val_row(val_v, r):
    z16f = jnp.zeros((16,), jnp.float32)
    for i in range(8):
        val_v[pl.ds(r * 128 + i * 16, 16)] = z16f


def _sc_segment_body(ei_hbm, he_hbm, out_hbm, ei_v, val_v, zbuf, acc,
                     sem_ld, sem_sc):
    c = lax.axis_index("c")
    s = lax.axis_index("s")
    w = s * 2 + c

    base = (w * _R) // _NW
    cnt = ((w + 1) * _R) // _NW - base
    load_base = jnp.minimum(base, _R - _LROWS)
    off = base - load_base

    ld_val = pltpu.async_copy(
        he_hbm.at[pl.ds(load_base * 128, _LROWS * 128)], val_v, sem_ld)
    ld_ei = pltpu.async_copy(ei_hbm.at[pl.ds(load_base, _HROWS)], ei_v,
                             sem_ld)

    z16f = jnp.zeros((16,), jnp.float32)

    def zero_body(i, carry):
        zbuf[pl.ds(i * 16, 16)] = z16f
        return carry

    lax.fori_loop(0, _CHUNK // 16, zero_body, 0)
    pltpu.sync_copy(zbuf, acc.at[pl.ds(s * _CHUNK, _CHUNK)])

    ld_val.wait()
    ld_ei.wait()

    @pl.when(off == 1)
    def _():
        _zero_val_row(val_v, 0)

    @pl.when(off + cnt <= _LROWS - 1)
    def _():
        _zero_val_row(val_v, _LROWS - 1)

    @pl.when(off + cnt <= _LROWS - 2)
    def _():
        _zero_val_row(val_v, _LROWS - 2)

    plsc.subcore_barrier()

    for h in range(2):
        if h == 1:
            pltpu.sync_copy(ei_hbm.at[pl.ds(load_base + _HROWS, _HROWS)],
                            ei_v)

        def group_body(g, carry, _h=h):
            descs = [
                pltpu.async_copy(
                    val_v.at[pl.ds((_h * _HROWS + g * 8 + i) * 128, 128)],
                    acc.at[ei_v.at[g * 8 + i, 1]], sem_sc, add=True)
                for i in range(8)
            ]
            for d in descs:
                d.wait()
            return carry

        lax.fori_loop(0, _HROWS // 8, group_body, 0)
        tail = [
            pltpu.async_copy(
                val_v.at[pl.ds((h * _HROWS + (_HROWS // 8) * 8 + i) * 128,
                               128)],
                acc.at[ei_v.at[(_HROWS // 8) * 8 + i, 1]], sem_sc, add=True)
            for i in range(_HROWS % 8)
        ]
        for d in tail:
            d.wait()
    plsc.subcore_barrier()

    pltpu.sync_copy(acc.at[pl.ds(s * _CHUNK, _CHUNK)], zbuf)
    pltpu.sync_copy(zbuf, out_hbm.at[c].at[pl.ds(s * _CHUNK, _CHUNK)])


@functools.cache
def _make_sc_segment():
    return pl.kernel(
        _sc_segment_body,
        out_type=jax.ShapeDtypeStruct((2, _NPAD), jnp.float32),
        mesh=plsc.VectorSubcoreMesh(core_axis_name="c", subcore_axis_name="s"),
        compiler_params=pltpu.CompilerParams(use_tc_tiling_on_sc=False),
        scratch_types=[
            pltpu.VMEM((_HROWS, 2, 128), jnp.int32),
            pltpu.VMEM((_LROWS * 128,), jnp.float32),
            pltpu.VMEM((_CHUNK,), jnp.float32),
            pltpu.VMEM_SHARED((_NPAD,), jnp.float32),
            pltpu.SemaphoreType.DMA,
            pltpu.SemaphoreType.DMA,
        ],
    )


_BN = 4096


def _mlp_a_body(hn_ref, w1a_ref, b1_ref, z_ref):
    z_ref[...] = (
        jnp.dot(hn_ref[...], w1a_ref[...], preferred_element_type=jnp.float32)
        + b1_ref[...]
    )


def _mlp_a(hn, w1a, b1):
    grid = (pl.cdiv(_N, _BN),)
    return pl.pallas_call(
        _mlp_a_body,
        grid=grid,
        in_specs=[
            pl.BlockSpec((_BN, _F), lambda i: (i, 0)),
            pl.BlockSpec((_F, _F), lambda i: (0, 0)),
            pl.BlockSpec((1, _F), lambda i: (0, 0)),
        ],
        out_specs=pl.BlockSpec((_BN, _F), lambda i: (i, 0)),
        out_shape=jax.ShapeDtypeStruct((_N, _F), jnp.float32),
    )(hn, w1a, b1)


def _mlp_b_body(pt_ref, z_ref, w1b_ref, w2_ref, b2_ref, w3t_ref, b3t_ref,
                out_ref):
    agg = pt_ref[...].reshape(_BN, 1)
    h = jnp.maximum(z_ref[...] + agg * w1b_ref[...], 0.0)
    h = jnp.dot(h, w2_ref[...], preferred_element_type=jnp.float32)
    h = jnp.maximum(h + b2_ref[...], 0.0)
    out_ref[...] = lax.dot_general(
        w3t_ref[...], h, (((1,), (1,)), ((), ())),
        preferred_element_type=jnp.float32,
    ) + b3t_ref[...]


def _mlp_b(pt, z, w1b, w2, b2, w3t, b3t):
    n_out = w3t.shape[0]
    grid = (pl.cdiv(_N, _BN),)
    return pl.pallas_call(
        _mlp_b_body,
        grid=grid,
        in_specs=[
            pl.BlockSpec((_BN,), lambda i: (i,)),
            pl.BlockSpec((_BN, _F), lambda i: (i, 0)),
            pl.BlockSpec((1, _F), lambda i: (0, 0)),
            pl.BlockSpec((_F, _F), lambda i: (0, 0)),
            pl.BlockSpec((1, _F), lambda i: (0, 0)),
            pl.BlockSpec((n_out, _F), lambda i: (0, 0)),
            pl.BlockSpec((n_out, 1), lambda i: (0, 0)),
        ],
        out_specs=pl.BlockSpec((n_out, _BN), lambda i: (0, i)),
        out_shape=jax.ShapeDtypeStruct((n_out, _N), jnp.float32),
    )(pt, z, w1b, w2, b2, w3t, b3t)


def kernel(hn, he, edge_index, W1, b1, W2, b2, W3, b3):
    ei3 = edge_index.astype(jnp.int32).reshape(2, _R, 128).transpose(1, 0, 2)
    he1 = he.reshape(_E)
    partials = _make_sc_segment()(ei3, he1)
    pt = partials[0, :_N] + partials[1, :_N]

    z = _mlp_a(hn, W1[:_F], b1.reshape(1, _F))
    out3 = _mlp_b(pt, z, W1[_F:_F + 1], W2, b2.reshape(1, _F),
                  W3.T, b3.reshape(-1, 1))
    return out3.T


# speedup vs baseline: 1.3460x; 1.1634x over previous
import functools

import jax
import jax.numpy as jnp
from jax import lax
from jax.experimental import pallas as pl
from jax.experimental.pallas import tpu as pltpu
from jax.experimental.pallas import tpu_sc as plsc

_N = 100000
_E = 1600000
_F = 128
_R = _E // 128
_NW = 32
_NPAD = 100096
_CHUNK = _NPAD // 16

_LROWS = 392
_HROWS = _LROWS // 2


def _zero_val_row(val_v, r):
    z16f = jnp.zeros((16,), jnp.float32)
    for i in range(8):
        val_v[pl.ds(r * 128 + i * 16, 16)] = z16f


def _sc_segment_body(ei_hbm, he_hbm, out_hbm, ei_v, val_v, zbuf, acc,
                     sem_ld, sem_sc):
    c = lax.axis_index("c")
    s = lax.axis_index("s")
    w = s * 2 + c

    base = (w * _R) // _NW
    cnt = ((w + 1) * _R) // _NW - base
    load_base = jnp.minimum(base, _R - _LROWS)
    off = base - load_base

    ld_val = pltpu.async_copy(
        he_hbm.at[pl.ds(load_base * 128, _LROWS * 128)], val_v, sem_ld)
    ld_ei = pltpu.async_copy(ei_hbm.at[pl.ds(load_base, _HROWS)], ei_v,
                             sem_ld)

    z16f = jnp.zeros((16,), jnp.float32)

    def zero_body(i, carry):
        zbuf[pl.ds(i * 16, 16)] = z16f
        return carry

    lax.fori_loop(0, _CHUNK // 16, zero_body, 0)
    pltpu.sync_copy(zbuf, acc.at[pl.ds(s * _CHUNK, _CHUNK)])

    ld_val.wait()
    ld_ei.wait()

    @pl.when(off == 1)
    def _():
        _zero_val_row(val_v, 0)

    @pl.when(off + cnt <= _LROWS - 1)
    def _():
        _zero_val_row(val_v, _LROWS - 1)

    @pl.when(off + cnt <= _LROWS - 2)
    def _():
        _zero_val_row(val_v, _LROWS - 2)

    plsc.subcore_barrier()

    _G = 7
    _NG = _HROWS // _G

    def _fire(g, _h):
        for i in range(_G):
            pltpu.async_copy(
                val_v.at[pl.ds((_h * _HROWS + g * _G + i) * 128, 128)],
                acc.at[ei_v.at[g * _G + i, 1]], sem_sc, add=True)

    def _drain_group():
        for _ in range(_G):
            pltpu.make_async_copy(he_hbm.at[pl.ds(0, 128)],
                                  val_v.at[pl.ds(0, 128)], sem_sc).wait()

    for h in range(2):
        if h == 1:
            pltpu.sync_copy(ei_hbm.at[pl.ds(load_base + _HROWS, _HROWS)],
                            ei_v)
        _fire(0, h)

        def group_body(g, carry, _h=h):
            _fire(g, _h)
            _drain_group()
            return carry

        lax.fori_loop(1, _NG, group_body, 0)
        _drain_group()
    plsc.subcore_barrier()

    pltpu.sync_copy(acc.at[pl.ds(s * _CHUNK, _CHUNK)], zbuf)
    pltpu.sync_copy(zbuf, out_hbm.at[c].at[pl.ds(s * _CHUNK, _CHUNK)])


@functools.cache
def _make_sc_segment():
    return pl.kernel(
        _sc_segment_body,
        out_type=jax.ShapeDtypeStruct((2, _NPAD), jnp.float32),
        mesh=plsc.VectorSubcoreMesh(core_axis_name="c", subcore_axis_name="s"),
        compiler_params=pltpu.CompilerParams(use_tc_tiling_on_sc=False),
        scratch_types=[
            pltpu.VMEM((_HROWS, 2, 128), jnp.int32),
            pltpu.VMEM((_LROWS * 128,), jnp.float32),
            pltpu.VMEM((_CHUNK,), jnp.float32),
            pltpu.VMEM_SHARED((_NPAD,), jnp.float32),
            pltpu.SemaphoreType.DMA,
            pltpu.SemaphoreType.DMA,
        ],
    )


_BN = 8192


def _mlp_a_body(hn_ref, w1a_ref, b1_ref, z_ref):
    z_ref[...] = (
        jnp.dot(hn_ref[...], w1a_ref[...], preferred_element_type=jnp.float32)
        + b1_ref[...]
    ).astype(jnp.bfloat16)


def _mlp_a(hn, w1a, b1):
    grid = (pl.cdiv(_N, _BN),)
    return pl.pallas_call(
        _mlp_a_body,
        grid=grid,
        in_specs=[
            pl.BlockSpec((_BN, _F), lambda i: (i, 0)),
            pl.BlockSpec((_F, _F), lambda i: (0, 0)),
            pl.BlockSpec((1, _F), lambda i: (0, 0)),
        ],
        out_specs=pl.BlockSpec((_BN, _F), lambda i: (i, 0)),
        out_shape=jax.ShapeDtypeStruct((_N, _F), jnp.bfloat16),
    )(hn, w1a, b1)


def _mlp_b_body(p_ref, z_ref, w1b_ref, w2_ref, b2_ref, w3t_ref, b3t_ref,
                out_ref):
    psum = p_ref[0:1, :] + p_ref[1:2, :]
    agg = psum.reshape(_BN, 1)
    h = jnp.maximum(z_ref[...].astype(jnp.float32) + agg * w1b_ref[...], 0.0)
    h = jnp.dot(h, w2_ref[...], preferred_element_type=jnp.float32)
    h = jnp.maximum(h + b2_ref[...], 0.0)
    out_ref[...] = lax.dot_general(
        w3t_ref[...], h, (((1,), (1,)), ((), ())),
        preferred_element_type=jnp.float32,
    ) + b3t_ref[...]


def _mlp_b(partials, z, w1b, w2, b2, w3t, b3t):
    n_out = w3t.shape[0]
    grid = (pl.cdiv(_N, _BN),)
    return pl.pallas_call(
        _mlp_b_body,
        grid=grid,
        in_specs=[
            pl.BlockSpec((2, _BN), lambda i: (0, i)),
            pl.BlockSpec((_BN, _F), lambda i: (i, 0)),
            pl.BlockSpec((1, _F), lambda i: (0, 0)),
            pl.BlockSpec((_F, _F), lambda i: (0, 0)),
            pl.BlockSpec((1, _F), lambda i: (0, 0)),
            pl.BlockSpec((n_out, _F), lambda i: (0, 0)),
            pl.BlockSpec((n_out, 1), lambda i: (0, 0)),
        ],
        out_specs=pl.BlockSpec((n_out, _BN), lambda i: (0, i)),
        out_shape=jax.ShapeDtypeStruct((n_out, _N), jnp.float32),
    )(partials, z, w1b, w2, b2, w3t, b3t)


def kernel(hn, he, edge_index, W1, b1, W2, b2, W3, b3):
    ei3 = edge_index.astype(jnp.int32).reshape(2, _R, 128).transpose(1, 0, 2)
    he1 = he.reshape(_E)
    partials = _make_sc_segment()(ei3, he1)

    z = _mlp_a(hn, W1[:_F], b1.reshape(1, _F))
    out3 = _mlp_b(partials, z, W1[_F:_F + 1], W2, b2.reshape(1, _F),
                  W3.T, b3.reshape(-1, 1))
    return out3.T
